# baseline (speedup 1.0000x reference)
import functools

import jax
import jax.numpy as jnp
from jax import lax
from jax.experimental import pallas as pl
from jax.experimental.pallas import tpu as pltpu
from jax.experimental.pallas import tpu_sc as plsc

_N = 10000
_E = 320000
_D = 128
_NP = 10240
_NW = 32
_CH = 128
_EPW = 10240
_NCHUNK = _EPW // _CH
_EP = _EPW * _NW
_RPS = _NP // 16

_mesh = plsc.VectorSubcoreMesh(core_axis_name="c", subcore_axis_name="s")


@functools.partial(
    pl.kernel,
    out_type=jax.ShapeDtypeStruct((2, _NP), jnp.float32),
    mesh=_mesh,
    scratch_types=[
        pltpu.VMEM((_NCHUNK, _CH), jnp.int32),
        pltpu.VMEM((_CH,), jnp.float32),
        pltpu.VMEM((_RPS,), jnp.float32),
        pltpu.VMEM_SHARED((_NP,), jnp.float32),
    ],
)
def _deg_call(dst_hbm, out_hbm, didx_v, ones_v, zeros_v, deg_sh):
    c = lax.axis_index("c")
    s = lax.axis_index("s")
    wid = s * 2 + c

    def fill_ones(i, carry):
        ones_v[pl.ds(i * 16, 16)] = jnp.full((16,), 1.0, jnp.float32)
        return carry

    lax.fori_loop(0, _CH // 16, fill_ones, 0)

    def fill_zeros(i, carry):
        zeros_v[pl.ds(i * 16, 16)] = jnp.zeros((16,), jnp.float32)
        return carry

    lax.fori_loop(0, _RPS // 16, fill_zeros, 0)

    pltpu.sync_copy(zeros_v, deg_sh.at[pl.ds(s * _RPS, _RPS)])
    pltpu.sync_copy(dst_hbm.at[pl.ds(wid * _NCHUNK, _NCHUNK)], didx_v)
    plsc.subcore_barrier()

    def body(i, carry):
        pltpu.sync_copy(ones_v, deg_sh.at[didx_v.at[i]], add=True)
        return carry

    lax.fori_loop(0, _NCHUNK, body, 0)
    plsc.subcore_barrier()
    pltpu.sync_copy(
        deg_sh.at[pl.ds(s * _RPS, _RPS)],
        out_hbm.at[c, pl.ds(s * _RPS, _RPS)],
    )


_NCHT = _EP // _CH
_LOPS = _NCHT // 16
_KQ = 32
_C1N = _LOPS - _KQ
_HD = _D // 2
_IDXW = (_LOPS + _KQ) * _CH


@functools.partial(
    pl.kernel,
    out_type=jax.ShapeDtypeStruct((2, 2, _NP, _HD), jnp.float32),
    mesh=_mesh,
    compiler_params=pltpu.CompilerParams(use_tc_tiling_on_sc=False),
    scratch_types=[
        pltpu.VMEM((_IDXW,), jnp.int32),
        pltpu.VMEM((_CH,), jnp.int32),
        pltpu.VMEM((_CH, _HD), jnp.float32),
        pltpu.VMEM((_CH, _HD), jnp.float32),
        pltpu.SemaphoreType.DMA,
        pltpu.VMEM_SHARED((_NP, _HD), jnp.float32),
        pltpu.VMEM_SHARED((_NP, _HD), jnp.float32),
    ],
)
def _agg_call(hv_hbm, slo_hbm, shi_hbm, src_hbm, dst_hbm, out_hbm,
              sidx_v, didx_v, rows0_v, rows1_v, gsem, agg_a, agg_b):
    c = lax.axis_index("c")
    s = lax.axis_index("s")
    rows = (rows0_v, rows1_v)
    rs = s * _RPS

    def zrow(r, carry):
        for j in range(_HD // 16):
            rows0_v[r, pl.ds(j * 16, 16)] = jnp.zeros((16,), jnp.float32)
        return carry

    lax.fori_loop(0, _CH, zrow, 0)

    def zero_core0():
        for k in range(_RPS // _CH):
            pltpu.sync_copy(rows0_v, agg_a.at[pl.ds(rs + k * _CH, _CH)])
            pltpu.sync_copy(rows0_v, agg_b.at[pl.ds(rs + k * _CH, _CH)])

    def zero_core1():
        for k in range(_RPS // _CH):
            pltpu.sync_copy(rows0_v, agg_a.at[pl.ds(rs + k * _CH, _CH)])
            pltpu.sync_copy(rows0_v, out_hbm.at[1, 0, pl.ds(rs + k * _CH, _CH)])
        def fill_idx(i, carry):
            sidx_v[pl.ds(i * 16, 16)] = (
                2 * (rs + i * 16) + 1 + 2 * lax.iota(jnp.int32, 16))
            return carry

        lax.fori_loop(0, _RPS // 16, fill_idx, 0)
        for k in range(_RPS // _CH):
            pltpu.async_copy(
                hv_hbm.at[sidx_v.at[pl.ds(k * _CH, _CH)]], rows1_v, gsem)
            pltpu.make_async_copy(
                hv_hbm.at[sidx_v.at[pl.ds(k * _CH, _CH)]], rows1_v, gsem).wait()
            pltpu.sync_copy(rows1_v, agg_b.at[pl.ds(rs + k * _CH, _CH)])

    pl.when(c == 0)(zero_core0)
    pl.when(c == 1)(zero_core1)
    plsc.subcore_barrier()

    def pipeline(idx_hbm, table, agg, base, n):
        eoff = pl.multiple_of(base * _CH, _CH)
        pltpu.sync_copy(idx_hbm.at[pl.ds(eoff, n * _CH)],
                        sidx_v.at[pl.ds(0, n * _CH)])

        def sidx(i):
            return sidx_v.at[pl.ds(pl.multiple_of(i * _CH, _CH), _CH)]

        def gather_start(i, buf):
            pltpu.async_copy(table.at[sidx(i)], buf, gsem)

        def gather_wait(i, buf):
            pltpu.make_async_copy(table.at[sidx(i)], buf, gsem).wait()

        def scatter(i, buf):
            doff = pl.multiple_of((base + i) * _CH, _CH)
            pltpu.sync_copy(dst_hbm.at[pl.ds(doff, _CH)], didx_v)
            pltpu.sync_copy(buf, agg.at[didx_v], add=True)

        gather_start(0, rows[0])

        def body(p, carry):
            for b in range(2):
                i = 2 * p + b
                gather_wait(i, rows[b])
                gather_start(i + 1, rows[1 - b])
                scatter(i, rows[b])
            return carry

        lax.fori_loop(0, n // 2 - 1, body, 0)

        i = n - 2
        gather_wait(i, rows[0])
        gather_start(i + 1, rows[1])
        scatter(i, rows[0])
        gather_wait(i + 1, rows[1])
        scatter(i + 1, rows[1])

    def work_core0():
        pipeline(slo_hbm, hv_hbm, agg_a, s * _LOPS, _LOPS)
        pipeline(shi_hbm, hv_hbm, agg_b, s * _KQ, _KQ)

    def work_core1():
        pipeline(src_hbm, agg_b, agg_a, 16 * _KQ + s * _C1N, _C1N)

    pl.when(c == 0)(work_core0)
    pl.when(c == 1)(work_core1)

    plsc.subcore_barrier()

    def flush_core0():
        pltpu.sync_copy(agg_a.at[pl.ds(rs, _RPS)],
                        out_hbm.at[0, 0, pl.ds(rs, _RPS)])
        pltpu.sync_copy(agg_b.at[pl.ds(rs, _RPS)],
                        out_hbm.at[0, 1, pl.ds(rs, _RPS)])

    def flush_core1():
        pltpu.sync_copy(agg_a.at[pl.ds(rs, _RPS)],
                        out_hbm.at[1, 1, pl.ds(rs, _RPS)])

    pl.when(c == 0)(flush_core0)
    pl.when(c == 1)(flush_core1)


_BLK = 1024
_GRID = _NP // _BLK


def _prep_body(x_ref, dv_ref, w_ref, o_ref):
    o_ref[...] = jnp.dot(
        x_ref[...] * dv_ref[...], w_ref[...],
        preferred_element_type=jnp.float32,
        precision=jax.lax.Precision.HIGHEST,
    )


_prep_call = pl.pallas_call(
    _prep_body,
    grid=(_GRID,),
    in_specs=[
        pl.BlockSpec((_BLK, _D), lambda i: (i, 0)),
        pl.BlockSpec((_BLK, _D), lambda i: (i, 0)),
        pl.BlockSpec((_D, _D), lambda i: (0, 0)),
    ],
    out_specs=pl.BlockSpec((_BLK, _D), lambda i: (i, 0)),
    out_shape=jax.ShapeDtypeStruct((_NP, _D), jnp.float32),
)


def _post1_body(p_ref, dv_ref, w_ref, h_ref, g_ref):
    dv = dv_ref[...]
    h = jnp.maximum((p_ref[0] + p_ref[1]) * dv, 0.0)
    h_ref[...] = h
    g_ref[...] = jnp.dot(
        h * dv, w_ref[...],
        preferred_element_type=jnp.float32,
        precision=jax.lax.Precision.HIGHEST,
    )


_post1_call = pl.pallas_call(
    _post1_body,
    grid=(_GRID,),
    in_specs=[
        pl.BlockSpec((2, _BLK, _D), lambda i: (0, i, 0)),
        pl.BlockSpec((_BLK, _D), lambda i: (i, 0)),
        pl.BlockSpec((_D, _D), lambda i: (0, 0)),
    ],
    out_specs=[
        pl.BlockSpec((_BLK, _D), lambda i: (i, 0)),
        pl.BlockSpec((_BLK, _D), lambda i: (i, 0)),
    ],
    out_shape=[
        jax.ShapeDtypeStruct((_NP, _D), jnp.float32),
        jax.ShapeDtypeStruct((_NP, _D), jnp.float32),
    ],
)


def _post2_body(p_ref, dv_ref, h_ref):
    h_ref[...] = jnp.maximum((p_ref[0] + p_ref[1]) * dv_ref[...], 0.0)


_post2_call = pl.pallas_call(
    _post2_body,
    grid=(_GRID,),
    in_specs=[
        pl.BlockSpec((2, _BLK, _D), lambda i: (0, i, 0)),
        pl.BlockSpec((_BLK, _D), lambda i: (i, 0)),
    ],
    out_specs=pl.BlockSpec((_BLK, _D), lambda i: (i, 0)),
    out_shape=jax.ShapeDtypeStruct((_NP, _D), jnp.float32),
)


def kernel(x, edge_index, W1, W2):
    src = edge_index[0]
    dst = edge_index[1]

    x_p = jnp.zeros((_NP, _D), jnp.float32).at[:_N].set(x)
    pad = jnp.full((_EP - _E,), _NP - 1, jnp.int32)
    src_p = jnp.concatenate([src, pad])
    dst_p = jnp.concatenate([dst, pad])
    dst_2d = dst_p.reshape(_EP // _CH, _CH)

    degp = _deg_call(dst_2d)
    dinv = jax.lax.rsqrt(jnp.maximum(degp[0] + degp[1], 1.0))
    dinv_mat = jnp.broadcast_to(dinv[:, None], (_NP, _D))

    slo = src_p * 2
    shi = slo + 1

    def unshuffle(P4):
        return P4.transpose(0, 2, 1, 3).reshape(2, _NP, _D)

    g1 = _prep_call(x_p, dinv_mat, W1)
    P1 = unshuffle(_agg_call(g1.reshape(2 * _NP, _HD), slo, shi, src_p, dst_p))
    h1, g2 = _post1_call(P1, dinv_mat, W2)
    P2 = unshuffle(_agg_call(g2.reshape(2 * _NP, _HD), slo, shi, src_p, dst_p))
    h2 = _post2_call(P2, dinv_mat)

    return jnp.stack([x, h1[:_N], h2[:_N]], axis=0)


# speedup vs baseline: 1.7302x; 1.7302x over previous
import functools

import jax
import jax.numpy as jnp
from jax import lax
from jax.experimental import pallas as pl
from jax.experimental.pallas import tpu as pltpu
from jax.experimental.pallas import tpu_sc as plsc

_N = 10000
_E = 320000
_D = 128
_NP = 10240
_NW = 32
_CH = 128
_EPW = 10240
_NCHUNK = _EPW // _CH
_EP = _EPW * _NW
_RPS = _NP // 16
_HD = _D // 2
_KPS = (_EP // _CH) // 16

_mesh = plsc.VectorSubcoreMesh(core_axis_name="c", subcore_axis_name="s")


@functools.partial(
    pl.kernel,
    out_type=jax.ShapeDtypeStruct((2, _NP), jnp.float32),
    mesh=_mesh,
    scratch_types=[
        pltpu.VMEM((_NCHUNK, _CH), jnp.int32),
        pltpu.VMEM((_CH,), jnp.float32),
        pltpu.VMEM((_RPS,), jnp.float32),
        pltpu.VMEM_SHARED((_NP,), jnp.float32),
    ],
)
def _deg_call(dst_hbm, out_hbm, didx_v, ones_v, zeros_v, deg_sh):
    c = lax.axis_index("c")
    s = lax.axis_index("s")
    wid = s * 2 + c

    def fill_ones(i, carry):
        ones_v[pl.ds(i * 16, 16)] = jnp.full((16,), 1.0, jnp.float32)
        return carry

    lax.fori_loop(0, _CH // 16, fill_ones, 0)

    def fill_zeros(i, carry):
        zeros_v[pl.ds(i * 16, 16)] = jnp.zeros((16,), jnp.float32)
        return carry

    lax.fori_loop(0, _RPS // 16, fill_zeros, 0)

    pltpu.sync_copy(zeros_v, deg_sh.at[pl.ds(s * _RPS, _RPS)])
    pltpu.sync_copy(dst_hbm.at[pl.ds(wid * _NCHUNK, _NCHUNK)], didx_v)
    plsc.subcore_barrier()

    def body(i, carry):
        pltpu.sync_copy(ones_v, deg_sh.at[didx_v.at[i]], add=True)
        return carry

    lax.fori_loop(0, _NCHUNK, body, 0)
    plsc.subcore_barrier()
    pltpu.sync_copy(
        deg_sh.at[pl.ds(s * _RPS, _RPS)],
        out_hbm.at[c, pl.ds(s * _RPS, _RPS)],
    )


@functools.partial(
    pl.kernel,
    out_type=jax.ShapeDtypeStruct((2, _NP, _HD), jnp.float32),
    mesh=_mesh,
    compiler_params=pltpu.CompilerParams(use_tc_tiling_on_sc=False),
    scratch_types=[
        pltpu.VMEM((_KPS * _CH,), jnp.int32),
        pltpu.VMEM((_CH,), jnp.int32),
        pltpu.VMEM((_CH, _HD), jnp.float32),
        pltpu.VMEM((_CH, _HD), jnp.float32),
        pltpu.SemaphoreType.DMA,
        pltpu.VMEM_SHARED((_NP, _HD), jnp.float32),
        pltpu.VMEM_SHARED((_NP, _HD), jnp.float32),
    ],
)
def _agg_call(hv_hbm, src_hbm, dst_hbm, out_hbm,
              sidx_v, didx_v, rows0_v, rows1_v, gsem, agg_sh, tab_sh):
    c = lax.axis_index("c")
    s = lax.axis_index("s")
    rows = (rows0_v, rows1_v)
    rs = s * _RPS

    def zrow(r, carry):
        for j in range(_HD // 16):
            rows0_v[r, pl.ds(j * 16, 16)] = jnp.zeros((16,), jnp.float32)
        return carry

    lax.fori_loop(0, _CH, zrow, 0)
    for k in range(_RPS // _CH):
        pltpu.sync_copy(rows0_v, agg_sh.at[pl.ds(rs + k * _CH, _CH)])

    def fill_idx(i, carry):
        sidx_v[pl.ds(i * 16, 16)] = (
            2 * (rs + i * 16) + c + 2 * lax.iota(jnp.int32, 16))
        return carry

    lax.fori_loop(0, _RPS // 16, fill_idx, 0)
    for k in range(_RPS // _CH):
        pltpu.async_copy(
            hv_hbm.at[sidx_v.at[pl.ds(k * _CH, _CH)]], rows1_v, gsem)
        pltpu.make_async_copy(
            hv_hbm.at[sidx_v.at[pl.ds(k * _CH, _CH)]], rows1_v, gsem).wait()
        pltpu.sync_copy(rows1_v, tab_sh.at[pl.ds(rs + k * _CH, _CH)])
    plsc.subcore_barrier()

    base = s * _KPS
    pltpu.sync_copy(src_hbm.at[pl.ds(base * _CH, _KPS * _CH)], sidx_v)

    def sidx(i):
        return sidx_v.at[pl.ds(pl.multiple_of(i * _CH, _CH), _CH)]

    def gather_start(i, buf):
        pltpu.async_copy(tab_sh.at[sidx(i)], buf, gsem)

    def gather_wait(i, buf):
        pltpu.make_async_copy(tab_sh.at[sidx(i)], buf, gsem).wait()

    def scatter(i, buf):
        doff = pl.multiple_of((base + i) * _CH, _CH)
        pltpu.sync_copy(dst_hbm.at[pl.ds(doff, _CH)], didx_v)
        pltpu.sync_copy(buf, agg_sh.at[didx_v], add=True)

    gather_start(0, rows[0])

    def body(p, carry):
        for b in range(2):
            i = 2 * p + b
            gather_wait(i, rows[b])
            gather_start(i + 1, rows[1 - b])
            scatter(i, rows[b])
        return carry

    lax.fori_loop(0, _KPS // 2 - 1, body, 0)

    i = _KPS - 2
    gather_wait(i, rows[0])
    gather_start(i + 1, rows[1])
    scatter(i, rows[0])
    gather_wait(i + 1, rows[1])
    scatter(i + 1, rows[1])

    plsc.subcore_barrier()
    pltpu.sync_copy(agg_sh.at[pl.ds(rs, _RPS)],
                    out_hbm.at[c, pl.ds(rs, _RPS)])


_BLK = 1024
_GRID = _NP // _BLK


def _prep_body(x_ref, dv_ref, w_ref, o_ref):
    o_ref[...] = jnp.dot(
        x_ref[...] * dv_ref[...], w_ref[...],
        preferred_element_type=jnp.float32,
        precision=jax.lax.Precision.HIGHEST,
    )


_prep_call = pl.pallas_call(
    _prep_body,
    grid=(_GRID,),
    in_specs=[
        pl.BlockSpec((_BLK, _D), lambda i: (i, 0)),
        pl.BlockSpec((_BLK, _D), lambda i: (i, 0)),
        pl.BlockSpec((_D, _D), lambda i: (0, 0)),
    ],
    out_specs=pl.BlockSpec((_BLK, _D), lambda i: (i, 0)),
    out_shape=jax.ShapeDtypeStruct((_NP, _D), jnp.float32),
)


def _post1_body(p_ref, dv_ref, w_ref, h_ref, g_ref):
    dv = dv_ref[...]
    h = jnp.maximum(p_ref[...] * dv, 0.0)
    h_ref[...] = h
    g_ref[...] = jnp.dot(
        h * dv, w_ref[...],
        preferred_element_type=jnp.float32,
        precision=jax.lax.Precision.HIGHEST,
    )


_post1_call = pl.pallas_call(
    _post1_body,
    grid=(_GRID,),
    in_specs=[
        pl.BlockSpec((_BLK, _D), lambda i: (i, 0)),
        pl.BlockSpec((_BLK, _D), lambda i: (i, 0)),
        pl.BlockSpec((_D, _D), lambda i: (0, 0)),
    ],
    out_specs=[
        pl.BlockSpec((_BLK, _D), lambda i: (i, 0)),
        pl.BlockSpec((_BLK, _D), lambda i: (i, 0)),
    ],
    out_shape=[
        jax.ShapeDtypeStruct((_NP, _D), jnp.float32),
        jax.ShapeDtypeStruct((_NP, _D), jnp.float32),
    ],
)


def _post2_body(p_ref, dv_ref, h_ref):
    h_ref[...] = jnp.maximum(p_ref[...] * dv_ref[...], 0.0)


_post2_call = pl.pallas_call(
    _post2_body,
    grid=(_GRID,),
    in_specs=[
        pl.BlockSpec((_BLK, _D), lambda i: (i, 0)),
        pl.BlockSpec((_BLK, _D), lambda i: (i, 0)),
    ],
    out_specs=pl.BlockSpec((_BLK, _D), lambda i: (i, 0)),
    out_shape=jax.ShapeDtypeStruct((_NP, _D), jnp.float32),
)


def kernel(x, edge_index, W1, W2):
    src = edge_index[0]
    dst = edge_index[1]

    x_p = jnp.zeros((_NP, _D), jnp.float32).at[:_N].set(x)
    pad = jnp.full((_EP - _E,), _NP - 1, jnp.int32)
    src_p = jnp.concatenate([src, pad])
    dst_p = jnp.concatenate([dst, pad])
    dst_2d = dst_p.reshape(_EP // _CH, _CH)

    degp = _deg_call(dst_2d)
    dinv = jax.lax.rsqrt(jnp.maximum(degp[0] + degp[1], 1.0))
    dinv_mat = jnp.broadcast_to(dinv[:, None], (_NP, _D))

    def unshuffle(Ph):
        return Ph.transpose(1, 0, 2).reshape(_NP, _D)

    g1 = _prep_call(x_p, dinv_mat, W1)
    P1 = unshuffle(_agg_call(g1.reshape(2 * _NP, _HD), src_p, dst_p))
    h1, g2 = _post1_call(P1, dinv_mat, W2)
    P2 = unshuffle(_agg_call(g2.reshape(2 * _NP, _HD), src_p, dst_p))
    h2 = _post2_call(P2, dinv_mat)

    return jnp.stack([x, h1[:_N], h2[:_N]], axis=0)
